# baseline (speedup 1.0000x reference)
import functools

import jax
import jax.numpy as jnp
from jax import lax
from jax.experimental import pallas as pl
from jax.experimental.pallas import tpu as pltpu
from jax.experimental.pallas import tpu_sc as plsc

N_COMP = 8192
EMB_DIM = 256

TM = 1024
TK = 256
_SEG_STARTS = (0, 11, 22)
_SEG_ENDS = (10, 21, 31)
_NK = N_COMP // TK


def _bf16_round(v):
    return v.astype(jnp.bfloat16).astype(jnp.float32)


def _argmin_body(x2_ref, e2_ref, x_ref, emb_ref, idx_ref,
                 sv_ref, si_ref, gv_ref, gi_ref):
    k = pl.program_id(1)
    xt = x_ref[0]
    et = emb_ref[...]
    x2 = x2_ref[0]
    e2 = e2_ref[...]
    s = jnp.dot(xt, et, preferred_element_type=jnp.float32)
    d = (x2 + e2) - 2.0 * s
    lmin = jnp.min(d, axis=1, keepdims=True)
    iota = lax.broadcasted_iota(jnp.int32, d.shape, 1)
    larg = jnp.min(jnp.where(d == lmin, iota, TK), axis=1, keepdims=True)
    larg = larg + k * TK

    is_seg_start = (k == _SEG_STARTS[0]) | (k == _SEG_STARTS[1]) \
        | (k == _SEG_STARTS[2])

    @pl.when(is_seg_start)
    def _():
        sv_ref[...] = lmin
        si_ref[...] = larg

    @pl.when(jnp.logical_not(is_seg_start))
    def _():
        sv, si = sv_ref[...], si_ref[...]
        better = (lmin < sv) | ((lmin == sv) & (larg < si))
        sv_ref[...] = jnp.where(better, lmin, sv)
        si_ref[...] = jnp.where(better, larg, si)

    @pl.when(k == _SEG_ENDS[0])
    def _():
        gv_ref[...] = _bf16_round(sv_ref[...])
        gi_ref[...] = si_ref[...]

    for end in _SEG_ENDS[1:]:
        @pl.when(k == end)
        def _():
            sv, si = sv_ref[...], si_ref[...]
            gv, gi = gv_ref[...], gi_ref[...]
            better = (sv < gv) | ((sv == gv) & (si < gi))
            gv_ref[...] = _bf16_round(jnp.where(better, sv, gv))
            gi_ref[...] = jnp.where(better, si, gi)

    @pl.when(k == _NK - 1)
    def _():
        idx_ref[0] = gi_ref[...]


def _argmin_indices(x, embeddings, x2, e2, interpret=False):
    nb = x.shape[0]
    return pl.pallas_call(
        _argmin_body,
        grid=(nb, _NK),
        in_specs=[
            pl.BlockSpec((1, TM, 1), lambda m, k: (m, 0, 0)),
            pl.BlockSpec((1, TK), lambda m, k: (0, k)),
            pl.BlockSpec((1, TM, EMB_DIM), lambda m, k: (m, 0, 0)),
            pl.BlockSpec((EMB_DIM, TK), lambda m, k: (0, k)),
        ],
        out_specs=pl.BlockSpec((1, TM, 1), lambda m, k: (m, 0, 0)),
        out_shape=jax.ShapeDtypeStruct((nb, TM, 1), jnp.int32),
        scratch_shapes=[
            pltpu.VMEM((TM, 1), jnp.float32),
            pltpu.VMEM((TM, 1), jnp.int32),
            pltpu.VMEM((TM, 1), jnp.float32),
            pltpu.VMEM((TM, 1), jnp.int32),
        ],
        interpret=interpret,
    )(x2, e2, x, embeddings)


def _sc_gather(table, idx):
    b, d = idx.shape[0], table.shape[1]
    info = plsc.get_sparse_core_info()
    nw = info.num_cores * info.num_subcores
    b_per_w = b // nw
    chunk = 256
    mesh = plsc.VectorSubcoreMesh(core_axis_name="c", subcore_axis_name="s")

    @functools.partial(
        pl.kernel, mesh=mesh,
        out_type=jax.ShapeDtypeStruct((b, d), jnp.float32),
        scratch_types=[
            pltpu.VMEM((chunk,), jnp.int32),
            pltpu.VMEM((chunk, d), jnp.float32),
            pltpu.SemaphoreType.DMA,
        ],
    )
    def k(table_hbm, idx_hbm, out_hbm, idx_v, rows_v, sem):
        wid = lax.axis_index("s") * info.num_cores + lax.axis_index("c")
        base = wid * b_per_w
        for c in range(b_per_w // chunk):
            off = base + c * chunk
            pltpu.sync_copy(idx_hbm.at[pl.ds(off, chunk)], idx_v)
            pltpu.async_copy(table_hbm.at[idx_v], rows_v, sem).wait()
            pltpu.sync_copy(rows_v, out_hbm.at[pl.ds(off, chunk)])

    return k(table, idx)


def kernel(x, embeddings):
    input_shape = x.shape
    flat = x.reshape(-1, EMB_DIM)
    x2 = jnp.sum(flat ** 2, axis=1).reshape(x.shape[0], TM, 1)
    e2 = jnp.sum(embeddings ** 2, axis=0).reshape(1, N_COMP)
    idx = _argmin_indices(x, embeddings, x2, e2)
    quantized = _sc_gather(embeddings.T, idx.reshape(-1))
    quantized = quantized.reshape(input_shape)
    return x + lax.stop_gradient(quantized - x)


# speedup vs baseline: 1.3724x; 1.3724x over previous
import functools

import jax
import jax.numpy as jnp
from jax import lax
from jax.experimental import pallas as pl
from jax.experimental.pallas import tpu as pltpu
from jax.experimental.pallas import tpu_sc as plsc

N_COMP = 8192
EMB_DIM = 256

TML = 1024
TKS = 256
_NK = N_COMP // TKS
_SEG_STARTS = (0, 11, 22)
_SEG_ENDS = (10, 21, 31)


def _bf16_round(v):
    return v.astype(jnp.bfloat16).astype(jnp.float32)


def _chain_body(x2_ref, e2_ref, eT_ref, xT_ref, idx_ref,
                sv_ref, si_ref, gv_ref, gi_ref):
    k = pl.program_id(1)
    s = jnp.dot(eT_ref[...], xT_ref[...],
                preferred_element_type=jnp.float32)
    x2 = x2_ref[...]
    e2 = e2_ref[...]
    base = k * TKS

    is_start = (k == _SEG_STARTS[0]) | (k == _SEG_STARTS[1]) \
        | (k == _SEG_STARTS[2])

    @pl.when(k == 0)
    def _():
        gv_ref[...] = jnp.full((1, TML), jnp.inf, jnp.float32)
        gi_ref[...] = jnp.zeros((1, TML), jnp.int32)

    inf8 = jnp.full((8, TML), jnp.inf, jnp.float32)
    zero8 = jnp.zeros((8, TML), jnp.int32)
    sv = jnp.where(is_start, inf8, sv_ref[...])
    si = jnp.where(is_start, zero8, si_ref[...])
    sub_iota = lax.broadcasted_iota(jnp.int32, (8, TML), 0)
    for j in range(TKS // 8):
        dj = (x2 + e2[8 * j:8 * j + 8, :]) - 2.0 * s[8 * j:8 * j + 8, :]
        ni = sub_iota + (base + 8 * j)
        better = dj < sv
        sv = jnp.where(better, dj, sv)
        si = jnp.where(better, ni, si)
    sv_ref[...] = sv
    si_ref[...] = si

    is_end = (k == _SEG_ENDS[0]) | (k == _SEG_ENDS[1]) | (k == _SEG_ENDS[2])

    @pl.when(is_end)
    def _():
        tv, ti = sv, si
        for h in (4, 2, 1):
            av, bv = tv[:h], tv[h:2 * h]
            ai, bi = ti[:h], ti[h:2 * h]
            bw = (bv < av) | ((bv == av) & (bi < ai))
            tv = jnp.where(bw, bv, av)
            ti = jnp.where(bw, bi, ai)
        gv, gi = gv_ref[...], gi_ref[...]
        better = (tv < gv) | ((tv == gv) & (ti < gi))
        gv_ref[...] = _bf16_round(jnp.where(better, tv, gv))
        gi_ref[...] = jnp.where(better, ti, gi)

    @pl.when(k == _NK - 1)
    def _():
        idx_ref[0] = gi_ref[...]


def _argmin_indices(xT, eT, x2, e2, interpret=False):
    nm = xT.shape[1] // TML
    return pl.pallas_call(
        _chain_body,
        grid=(nm, _NK),
        in_specs=[
            pl.BlockSpec((1, TML), lambda m, k: (0, m)),
            pl.BlockSpec((TKS, 1), lambda m, k: (k, 0)),
            pl.BlockSpec((TKS, EMB_DIM), lambda m, k: (k, 0)),
            pl.BlockSpec((EMB_DIM, TML), lambda m, k: (0, m)),
        ],
        out_specs=pl.BlockSpec((1, 1, TML), lambda m, k: (m, 0, 0)),
        out_shape=jax.ShapeDtypeStruct((nm, 1, TML), jnp.int32),
        scratch_shapes=[
            pltpu.VMEM((8, TML), jnp.float32),
            pltpu.VMEM((8, TML), jnp.int32),
            pltpu.VMEM((1, TML), jnp.float32),
            pltpu.VMEM((1, TML), jnp.int32),
        ],
        interpret=interpret,
    )(x2, e2, eT, xT)


def _sc_gather(table, idx):
    b, d = idx.shape[0], table.shape[1]
    info = plsc.get_sparse_core_info()
    nw = info.num_cores * info.num_subcores
    b_per_w = b // nw
    chunk = 256
    mesh = plsc.VectorSubcoreMesh(core_axis_name="c", subcore_axis_name="s")

    @functools.partial(
        pl.kernel, mesh=mesh,
        out_type=jax.ShapeDtypeStruct((b, d), jnp.float32),
        scratch_types=[
            pltpu.VMEM((chunk,), jnp.int32),
            pltpu.VMEM((chunk, d), jnp.float32),
            pltpu.SemaphoreType.DMA,
        ],
    )
    def k(table_hbm, idx_hbm, out_hbm, idx_v, rows_v, sem):
        wid = lax.axis_index("s") * info.num_cores + lax.axis_index("c")
        base = wid * b_per_w
        for c in range(b_per_w // chunk):
            off = base + c * chunk
            pltpu.sync_copy(idx_hbm.at[pl.ds(off, chunk)], idx_v)
            pltpu.async_copy(table_hbm.at[idx_v], rows_v, sem).wait()
            pltpu.sync_copy(rows_v, out_hbm.at[pl.ds(off, chunk)])

    return k(table, idx)


def kernel(x, embeddings):
    input_shape = x.shape
    flat = x.reshape(-1, EMB_DIM)
    x2 = jnp.sum(flat ** 2, axis=1).reshape(1, -1)
    e2 = jnp.sum(embeddings ** 2, axis=0).reshape(N_COMP, 1)
    xT = flat.T
    eT = embeddings.T
    idx = _argmin_indices(xT, eT, x2, e2)
    quantized = _sc_gather(eT, idx.reshape(-1))
    quantized = quantized.reshape(input_shape)
    return x + lax.stop_gradient(quantized - x)


# speedup vs baseline: 1.7584x; 1.2812x over previous
import functools

import jax
import jax.numpy as jnp
from jax import lax
from jax.experimental import pallas as pl
from jax.experimental.pallas import tpu as pltpu
from jax.experimental.pallas import tpu_sc as plsc

N_COMP = 8192
EMB_DIM = 256

TML = 2048
TKS = 256
CW = 32
_NK = N_COMP // TKS
_SEG_STARTS = (0, 11, 22)
_SEG_ENDS = (10, 21, 31)


def _bf16_round(v):
    return v.astype(jnp.bfloat16).astype(jnp.float32)


def _chain_body(x2_ref, e2_ref, eT_ref, xT_ref, idx_ref,
                sv_ref, si_ref, gv_ref, gi_ref):
    k = pl.program_id(1)
    s = jnp.dot(eT_ref[...], xT_ref[...],
                preferred_element_type=jnp.float32)
    x2 = x2_ref[...]
    e2 = e2_ref[...]
    base = k * TKS

    is_start = (k == _SEG_STARTS[0]) | (k == _SEG_STARTS[1]) \
        | (k == _SEG_STARTS[2])

    @pl.when(k == 0)
    def _():
        gv_ref[...] = jnp.full((1, TML), jnp.inf, jnp.float32)
        gi_ref[...] = jnp.zeros((1, TML), jnp.int32)

    infc = jnp.full((CW, TML), jnp.inf, jnp.float32)
    zeroc = jnp.zeros((CW, TML), jnp.int32)
    sv = jnp.where(is_start, infc, sv_ref[...])
    si = jnp.where(is_start, zeroc, si_ref[...])
    sub_iota = lax.broadcasted_iota(jnp.int32, (CW, TML), 0)
    for j in range(TKS // CW):
        dj = (x2 + e2[CW * j:CW * j + CW, :]) - 2.0 * s[CW * j:CW * j + CW, :]
        ni = sub_iota + (base + CW * j)
        better = dj < sv
        sv = jnp.where(better, dj, sv)
        si = jnp.where(better, ni, si)
    sv_ref[...] = sv
    si_ref[...] = si

    is_end = (k == _SEG_ENDS[0]) | (k == _SEG_ENDS[1]) | (k == _SEG_ENDS[2])

    @pl.when(is_end)
    def _():
        tv, ti = sv, si
        for h in (16, 8, 4, 2, 1):
            av, bv = tv[:h], tv[h:2 * h]
            ai, bi = ti[:h], ti[h:2 * h]
            bw = (bv < av) | ((bv == av) & (bi < ai))
            tv = jnp.where(bw, bv, av)
            ti = jnp.where(bw, bi, ai)
        gv, gi = gv_ref[...], gi_ref[...]
        better = (tv < gv) | ((tv == gv) & (ti < gi))
        gv_ref[...] = _bf16_round(jnp.where(better, tv, gv))
        gi_ref[...] = jnp.where(better, ti, gi)

    @pl.when(k == _NK - 1)
    def _():
        idx_ref[0] = gi_ref[...]


def _argmin_indices(xT, eT, x2, e2, interpret=False):
    nm = xT.shape[1] // TML
    return pl.pallas_call(
        _chain_body,
        grid=(nm, _NK),
        in_specs=[
            pl.BlockSpec((1, TML), lambda m, k: (0, m)),
            pl.BlockSpec((TKS, 1), lambda m, k: (k, 0)),
            pl.BlockSpec((TKS, EMB_DIM), lambda m, k: (k, 0)),
            pl.BlockSpec((EMB_DIM, TML), lambda m, k: (0, m)),
        ],
        out_specs=pl.BlockSpec((1, 1, TML), lambda m, k: (m, 0, 0)),
        out_shape=jax.ShapeDtypeStruct((nm, 1, TML), jnp.int32),
        scratch_shapes=[
            pltpu.VMEM((CW, TML), jnp.float32),
            pltpu.VMEM((CW, TML), jnp.int32),
            pltpu.VMEM((1, TML), jnp.float32),
            pltpu.VMEM((1, TML), jnp.int32),
        ],
        interpret=interpret,
    )(x2, e2, eT, xT)


def _sc_gather(table, idx):
    b, d = idx.shape[0], table.shape[1]
    info = plsc.get_sparse_core_info()
    nw = info.num_cores * info.num_subcores
    b_per_w = b // nw
    chunk = 256
    mesh = plsc.VectorSubcoreMesh(core_axis_name="c", subcore_axis_name="s")

    @functools.partial(
        pl.kernel, mesh=mesh,
        out_type=jax.ShapeDtypeStruct((b, d), jnp.float32),
        scratch_types=[
            pltpu.VMEM((chunk,), jnp.int32),
            pltpu.VMEM((chunk, d), jnp.float32),
            pltpu.SemaphoreType.DMA,
        ],
    )
    def k(table_hbm, idx_hbm, out_hbm, idx_v, rows_v, sem):
        wid = lax.axis_index("s") * info.num_cores + lax.axis_index("c")
        base = wid * b_per_w
        for c in range(b_per_w // chunk):
            off = base + c * chunk
            pltpu.sync_copy(idx_hbm.at[pl.ds(off, chunk)], idx_v)
            pltpu.async_copy(table_hbm.at[idx_v], rows_v, sem).wait()
            pltpu.sync_copy(rows_v, out_hbm.at[pl.ds(off, chunk)])

    return k(table, idx)


def kernel(x, embeddings):
    input_shape = x.shape
    flat = x.reshape(-1, EMB_DIM)
    x2 = jnp.sum(flat ** 2, axis=1).reshape(1, -1)
    e2 = jnp.sum(embeddings ** 2, axis=0).reshape(N_COMP, 1)
    xT = flat.T
    eT = embeddings.T
    idx = _argmin_indices(xT, eT, x2, e2)
    quantized = _sc_gather(eT, idx.reshape(-1))
    quantized = quantized.reshape(input_shape)
    return x + lax.stop_gradient(quantized - x)


# speedup vs baseline: 1.8317x; 1.0417x over previous
import functools

import jax
import jax.numpy as jnp
from jax import lax
from jax.experimental import pallas as pl
from jax.experimental.pallas import tpu as pltpu
from jax.experimental.pallas import tpu_sc as plsc

N_COMP = 8192
EMB_DIM = 256

TML = 2048
TKS = 256
CW = 64
_NK = N_COMP // TKS
_SEG_STARTS = (0, 11, 22)
_SEG_ENDS = (10, 21, 31)


def _bf16_round(v):
    return v.astype(jnp.bfloat16).astype(jnp.float32)


def _chain_body(x2_ref, e2_ref, eT_ref, xT_ref, idx_ref,
                sv_ref, si_ref, gv_ref, gi_ref):
    k = pl.program_id(1)
    s2 = jnp.dot(eT_ref[...] * 2.0, xT_ref[...],
                 preferred_element_type=jnp.float32)
    x2 = x2_ref[...]
    e2 = e2_ref[...]
    base = k * TKS

    is_start = (k == _SEG_STARTS[0]) | (k == _SEG_STARTS[1]) \
        | (k == _SEG_STARTS[2])

    @pl.when(k == 0)
    def _():
        gv_ref[...] = jnp.full((1, TML), jnp.inf, jnp.float32)
        gi_ref[...] = jnp.zeros((1, TML), jnp.int32)

    infc = jnp.full((CW, TML), jnp.inf, jnp.float32)
    zeroc = jnp.zeros((CW, TML), jnp.int32)
    sv = jnp.where(is_start, infc, sv_ref[...])
    si = jnp.where(is_start, zeroc, si_ref[...])
    sub_iota = lax.broadcasted_iota(jnp.int32, (CW, TML), 0)
    for j in range(TKS // CW):
        dj = (x2 + e2[CW * j:CW * j + CW, :]) - s2[CW * j:CW * j + CW, :]
        ni = sub_iota + (base + CW * j)
        better = dj < sv
        sv = jnp.where(better, dj, sv)
        si = jnp.where(better, ni, si)
    sv_ref[...] = sv
    si_ref[...] = si

    is_end = (k == _SEG_ENDS[0]) | (k == _SEG_ENDS[1]) | (k == _SEG_ENDS[2])

    @pl.when(is_end)
    def _():
        tv, ti = sv, si
        for h in (32, 16, 8, 4, 2, 1):
            av, bv = tv[:h], tv[h:2 * h]
            ai, bi = ti[:h], ti[h:2 * h]
            bw = (bv < av) | ((bv == av) & (bi < ai))
            tv = jnp.where(bw, bv, av)
            ti = jnp.where(bw, bi, ai)
        gv, gi = gv_ref[...], gi_ref[...]
        better = (tv < gv) | ((tv == gv) & (ti < gi))
        gv_ref[...] = _bf16_round(jnp.where(better, tv, gv))
        gi_ref[...] = jnp.where(better, ti, gi)

    @pl.when(k == _NK - 1)
    def _():
        idx_ref[0] = gi_ref[...]


def _argmin_indices(xT, eT, x2, e2, interpret=False):
    nm = xT.shape[1] // TML
    return pl.pallas_call(
        _chain_body,
        grid=(nm, _NK),
        in_specs=[
            pl.BlockSpec((1, TML), lambda m, k: (0, m)),
            pl.BlockSpec((TKS, 1), lambda m, k: (k, 0)),
            pl.BlockSpec((TKS, EMB_DIM), lambda m, k: (k, 0)),
            pl.BlockSpec((EMB_DIM, TML), lambda m, k: (0, m)),
        ],
        out_specs=pl.BlockSpec((1, 1, TML), lambda m, k: (m, 0, 0)),
        out_shape=jax.ShapeDtypeStruct((nm, 1, TML), jnp.int32),
        scratch_shapes=[
            pltpu.VMEM((CW, TML), jnp.float32),
            pltpu.VMEM((CW, TML), jnp.int32),
            pltpu.VMEM((1, TML), jnp.float32),
            pltpu.VMEM((1, TML), jnp.int32),
        ],
        interpret=interpret,
    )(x2, e2, eT, xT)


def _sc_gather(table, idx):
    b, d = idx.shape[0], table.shape[1]
    info = plsc.get_sparse_core_info()
    nw = info.num_cores * info.num_subcores
    b_per_w = b // nw
    chunk = 256
    mesh = plsc.VectorSubcoreMesh(core_axis_name="c", subcore_axis_name="s")

    @functools.partial(
        pl.kernel, mesh=mesh,
        out_type=jax.ShapeDtypeStruct((b, d), jnp.float32),
        scratch_types=[
            pltpu.VMEM((chunk,), jnp.int32),
            pltpu.VMEM((chunk, d), jnp.float32),
            pltpu.SemaphoreType.DMA,
        ],
    )
    def k(table_hbm, idx_hbm, out_hbm, idx_v, rows_v, sem):
        wid = lax.axis_index("s") * info.num_cores + lax.axis_index("c")
        base = wid * b_per_w
        for c in range(b_per_w // chunk):
            off = base + c * chunk
            pltpu.sync_copy(idx_hbm.at[pl.ds(off, chunk)], idx_v)
            pltpu.async_copy(table_hbm.at[idx_v], rows_v, sem).wait()
            pltpu.sync_copy(rows_v, out_hbm.at[pl.ds(off, chunk)])

    return k(table, idx)


def kernel(x, embeddings):
    input_shape = x.shape
    flat = x.reshape(-1, EMB_DIM)
    x2 = jnp.sum(flat ** 2, axis=1).reshape(1, -1)
    e2 = jnp.sum(embeddings ** 2, axis=0).reshape(N_COMP, 1)
    xT = flat.T
    eT = embeddings.T
    idx = _argmin_indices(xT, eT, x2, e2)
    quantized = _sc_gather(eT, idx.reshape(-1))
    quantized = quantized.reshape(input_shape)
    return x + lax.stop_gradient(quantized - x)


# speedup vs baseline: 3.0173x; 1.6473x over previous
import functools

import jax
import jax.numpy as jnp
from jax import lax
from jax.experimental import pallas as pl
from jax.experimental.pallas import tpu as pltpu
from jax.experimental.pallas import tpu_sc as plsc

N_COMP = 8192
EMB_DIM = 256

TML = 2048
TKS = 256
CW = 64
_NK = N_COMP // TKS
_SEG_STARTS = (0, 11, 22)
_SEG_ENDS = (10, 21, 31)


def _bf16_round(v):
    return v.astype(jnp.bfloat16).astype(jnp.float32)


def _chain_body(x2_ref, e2_ref, eT_ref, xT_ref, idx_ref):
    x2 = x2_ref[...]
    xt = xT_ref[...]
    sub_iota = lax.broadcasted_iota(jnp.int32, (CW, TML), 0)
    gv = jnp.full((1, TML), jnp.inf, jnp.float32)
    gi = jnp.zeros((1, TML), jnp.int32)
    for lo_t, hi_t in ((0, 11), (11, 22), (22, _NK)):
        sv = jnp.full((CW, TML), jnp.inf, jnp.float32)
        si = jnp.zeros((CW, TML), jnp.int32)
        for k in range(lo_t, hi_t):
            s2 = jnp.dot(eT_ref[k * TKS:(k + 1) * TKS, :] * 2.0, xt,
                         preferred_element_type=jnp.float32)
            for j in range(TKS // CW):
                lo = k * TKS + CW * j
                dj = (x2 + e2_ref[lo:lo + CW, :]) - s2[CW * j:CW * j + CW, :]
                ni = sub_iota + lo
                better = dj < sv
                sv = jnp.where(better, dj, sv)
                si = jnp.where(better, ni, si)
        tv, ti = sv, si
        for h in (32, 16, 8, 4, 2, 1):
            av, bv = tv[:h], tv[h:2 * h]
            ai, bi = ti[:h], ti[h:2 * h]
            bw = (bv < av) | ((bv == av) & (bi < ai))
            tv = jnp.where(bw, bv, av)
            ti = jnp.where(bw, bi, ai)
        better = (tv < gv) | ((tv == gv) & (ti < gi))
        gv = _bf16_round(jnp.where(better, tv, gv))
        gi = jnp.where(better, ti, gi)
    idx_ref[0] = gi


def _argmin_indices(xT, eT, x2, e2, interpret=False):
    nm = xT.shape[1] // TML
    return pl.pallas_call(
        _chain_body,
        grid=(nm,),
        in_specs=[
            pl.BlockSpec((1, TML), lambda m: (0, m)),
            pl.BlockSpec((N_COMP, 1), lambda m: (0, 0)),
            pl.BlockSpec((N_COMP, EMB_DIM), lambda m: (0, 0)),
            pl.BlockSpec((EMB_DIM, TML), lambda m: (0, m)),
        ],
        out_specs=pl.BlockSpec((1, 1, TML), lambda m: (m, 0, 0)),
        out_shape=jax.ShapeDtypeStruct((nm, 1, TML), jnp.int32),
        interpret=interpret,
    )(x2, e2, eT, xT)


def _sc_gather(table, idx):
    b, d = idx.shape[0], table.shape[1]
    info = plsc.get_sparse_core_info()
    nw = info.num_cores * info.num_subcores
    b_per_w = b // nw
    chunk = 256
    mesh = plsc.VectorSubcoreMesh(core_axis_name="c", subcore_axis_name="s")

    @functools.partial(
        pl.kernel, mesh=mesh,
        out_type=jax.ShapeDtypeStruct((b, d), jnp.float32),
        scratch_types=[
            pltpu.VMEM((chunk,), jnp.int32),
            pltpu.VMEM((chunk, d), jnp.float32),
            pltpu.SemaphoreType.DMA,
        ],
    )
    def k(table_hbm, idx_hbm, out_hbm, idx_v, rows_v, sem):
        wid = lax.axis_index("s") * info.num_cores + lax.axis_index("c")
        base = wid * b_per_w
        for c in range(b_per_w // chunk):
            off = base + c * chunk
            pltpu.sync_copy(idx_hbm.at[pl.ds(off, chunk)], idx_v)
            pltpu.async_copy(table_hbm.at[idx_v], rows_v, sem).wait()
            pltpu.sync_copy(rows_v, out_hbm.at[pl.ds(off, chunk)])

    return k(table, idx)


def kernel(x, embeddings):
    input_shape = x.shape
    flat = x.reshape(-1, EMB_DIM)
    x2 = jnp.sum(flat ** 2, axis=1).reshape(1, -1)
    e2 = jnp.sum(embeddings ** 2, axis=0).reshape(N_COMP, 1)
    xT = flat.T
    eT = embeddings.T
    idx = _argmin_indices(xT, eT, x2, e2)
    quantized = _sc_gather(eT, idx.reshape(-1))
    quantized = quantized.reshape(input_shape)
    return x + lax.stop_gradient(quantized - x)
